# Initial kernel scaffold; baseline (speedup 1.0000x reference)
import functools

import jax
import jax.numpy as jnp
from jax import lax
from jax.experimental import pallas as pl
from jax.experimental.pallas import tpu as pltpu
from jax.experimental.pallas import tpu_sc as plsc

N = 1024
NLANES = 16
NCORES = 2
NSUBCORES = 16
NW = NCORES * NSUBCORES
SPW = N // NW


def _make_sc_kernel(C, W):
    mesh = plsc.VectorSubcoreMesh(core_axis_name="c", subcore_axis_name="s")

    @functools.partial(
        pl.kernel,
        mesh=mesh,
        out_type=jax.ShapeDtypeStruct((N, C, W), jnp.float32),
        scratch_types=[
            pltpu.VMEM((SPW, 2, C), jnp.int32),
            pltpu.VMEM((SPW, 2, NLANES), jnp.float32),
            pltpu.VMEM((2, C, W), jnp.float32),
            pltpu.VMEM((2, C, W), jnp.float32),
            pltpu.VMEM((2, C, W), jnp.float32),
            pltpu.SemaphoreType.DMA,
            pltpu.SemaphoreType.DMA,
            pltpu.SemaphoreType.DMA,
        ],
    )
    def sc_kernel(table_hbm, idx_hbm, wts_hbm, out_hbm,
                  idx_v, w_v, buf0, buf1, obuf, sem0, sem1, osem):
        wid = lax.axis_index("s") * NCORES + lax.axis_index("c")
        base = wid * SPW
        pltpu.sync_copy(idx_hbm.at[pl.ds(base, SPW)], idx_v)
        pltpu.sync_copy(wts_hbm.at[pl.ds(base, SPW)], w_v)

        bufs = (buf0, buf1)
        sems = (sem0, sem1)

        def start_gather(s, par):
            a = pltpu.async_copy(
                table_hbm.at[idx_v.at[s, 0]], bufs[par].at[0], sems[par])
            b = pltpu.async_copy(
                table_hbm.at[idx_v.at[s, 1]], bufs[par].at[1], sems[par])
            return a, b

        def blend(s, par):
            buf = bufs[par]
            w0 = w_v[s, 0]
            w1 = w_v[s, 1]
            ob = obuf.at[par]

            def rbody(r, _):
                for cc in range(W // NLANES):
                    sl = pl.ds(cc * NLANES, NLANES)
                    a = buf[0, r, sl]
                    b = buf[1, r, sl]
                    ob[r, sl] = w0 * a + w1 * b
                return 0

            lax.fori_loop(0, C, rbody, 0)

        pend = start_gather(0, 0)
        out_pend = [None, None]
        for s in range(SPW):
            par = s % 2
            nxt = start_gather(s + 1, 1 - par) if s + 1 < SPW else None
            pend[0].wait()
            pend[1].wait()
            if out_pend[par] is not None:
                out_pend[par].wait()
            blend(s, par)
            oc = pltpu.async_copy(obuf.at[par], out_hbm.at[base + s], osem)
            out_pend[par] = oc
            pend = nxt
        for oc in out_pend:
            if oc is not None:
                oc.wait()

    return sc_kernel


def kernel(x, image_num, image_ids, rows):
    del image_num, image_ids
    NIMG, C, H, W = x.shape
    NROW = NIMG * H

    iy = rows - 0.5
    iy0 = jnp.floor(iy)
    w1 = iy - iy0
    w0 = 1.0 - w1
    y0 = iy0.astype(jnp.int32)
    y1 = y0 + 1
    w0 = jnp.where((y0 >= 0) & (y0 <= NROW - 1), 0.5 * w0, 0.0)
    w1 = jnp.where((y1 >= 0) & (y1 <= NROW - 1), 0.5 * w1, 0.0)
    y0c = jnp.clip(y0, 0, NROW - 1)
    y1c = jnp.clip(y1, 0, NROW - 1)

    table = x.reshape(NIMG * C * H, W)
    coffs = (jnp.arange(C, dtype=jnp.int32) * H)[None, None, :]
    base01 = jnp.stack(
        [(y0c // H) * (C * H) + (y0c % H), (y1c // H) * (C * H) + (y1c % H)],
        axis=1)
    idx = base01[:, :, None] + coffs
    wts = jnp.broadcast_to(
        jnp.stack([w0, w1], axis=1)[:, :, None], (N, 2, NLANES))

    return _make_sc_kernel(C, W)(table, idx, wts)


# speedup vs baseline: 3.1354x; 3.1354x over previous
import functools

import jax
import jax.numpy as jnp
from jax import lax
from jax.experimental import pallas as pl
from jax.experimental.pallas import tpu as pltpu
from jax.experimental.pallas import tpu_sc as plsc

N = 1024
NLANES = 16
NCORES = 2
NSUBCORES = 16
NW = NCORES * NSUBCORES
SPW = N // NW


def _make_sc_kernel(C, W):
    mesh = plsc.VectorSubcoreMesh(core_axis_name="c", subcore_axis_name="s")

    @functools.partial(
        pl.kernel,
        mesh=mesh,
        out_type=jax.ShapeDtypeStruct((N, C, W), jnp.float32),
        compiler_params=pltpu.CompilerParams(use_tc_tiling_on_sc=False),
        scratch_types=[
            pltpu.VMEM((SPW, 2, C), jnp.int32),
            pltpu.VMEM((SPW, 2, NLANES), jnp.float32),
            pltpu.VMEM((2, C, W), jnp.float32),
            pltpu.VMEM((2, C, W), jnp.float32),
            pltpu.VMEM((C, W), jnp.float32),
            pltpu.SemaphoreType.DMA,
            pltpu.SemaphoreType.DMA,
            pltpu.SemaphoreType.DMA,
        ],
    )
    def sc_kernel(table_hbm, idx_hbm, wts_hbm, out_hbm,
                  idx_v, w_v, buf0, buf1, obuf, sem0, sem1, osem):
        wid = lax.axis_index("s") * NCORES + lax.axis_index("c")
        base = wid * SPW
        pltpu.sync_copy(idx_hbm.at[pl.ds(base, SPW)], idx_v)
        pltpu.sync_copy(wts_hbm.at[pl.ds(base, SPW)], w_v)

        bufs = (buf0, buf1)
        sems = (sem0, sem1)

        def start_gather(s, par):
            a = pltpu.async_copy(
                table_hbm.at[idx_v.at[s, 0]], bufs[par].at[0], sems[par])
            b = pltpu.async_copy(
                table_hbm.at[idx_v.at[s, 1]], bufs[par].at[1], sems[par])
            return a, b

        def blend(s, par):
            buf = bufs[par]
            w0 = w_v[s, 0]
            w1 = w_v[s, 1]

            def rbody(r, _):
                for cc in range(W // NLANES):
                    sl = pl.ds(cc * NLANES, NLANES)
                    a = buf[0, r, sl]
                    b = buf[1, r, sl]
                    obuf[r, sl] = w0 * a + w1 * b
                return 0

            lax.fori_loop(0, C, rbody, 0)

        pend = start_gather(0, 0)
        out_pend = None
        for s in range(SPW):
            par = s % 2
            nxt = start_gather(s + 1, 1 - par) if s + 1 < SPW else None
            pend[0].wait()
            pend[1].wait()
            if out_pend is not None:
                out_pend.wait()
            blend(s, par)
            out_pend = pltpu.async_copy(obuf, out_hbm.at[base + s], osem)
            pend = nxt
        out_pend.wait()

    return sc_kernel


def kernel(x, image_num, image_ids, rows):
    del image_num, image_ids
    NIMG, C, H, W = x.shape
    NROW = NIMG * H

    iy = rows - 0.5
    iy0 = jnp.floor(iy)
    w1 = iy - iy0
    w0 = 1.0 - w1
    y0 = iy0.astype(jnp.int32)
    y1 = y0 + 1
    w0 = jnp.where((y0 >= 0) & (y0 <= NROW - 1), 0.5 * w0, 0.0)
    w1 = jnp.where((y1 >= 0) & (y1 <= NROW - 1), 0.5 * w1, 0.0)
    y0c = jnp.clip(y0, 0, NROW - 1)
    y1c = jnp.clip(y1, 0, NROW - 1)

    table = x.reshape(NIMG * C * H, W)
    coffs = (jnp.arange(C, dtype=jnp.int32) * H)[None, None, :]
    base01 = jnp.stack(
        [(y0c // H) * (C * H) + (y0c % H), (y1c // H) * (C * H) + (y1c % H)],
        axis=1)
    idx = base01[:, :, None] + coffs
    wts = jnp.broadcast_to(
        jnp.stack([w0, w1], axis=1)[:, :, None], (N, 2, NLANES))

    return _make_sc_kernel(C, W)(table, idx, wts)


# speedup vs baseline: 4.7447x; 1.5133x over previous
import functools

import jax
import jax.numpy as jnp
from jax import lax
from jax.experimental import pallas as pl
from jax.experimental.pallas import tpu as pltpu
from jax.experimental.pallas import tpu_sc as plsc

N = 1024
NLANES = 16
NCORES = 2
NSUBCORES = 16
NW = NCORES * NSUBCORES
SPW = N // NW
CSPLIT = 2
NBUF = 4


def _make_sc_kernel(NIMG, C, H, W):
    CH = C // CSPLIT
    NT = SPW * CSPLIT
    mesh = plsc.VectorSubcoreMesh(core_axis_name="c", subcore_axis_name="s")

    @functools.partial(
        pl.kernel,
        mesh=mesh,
        out_type=jax.ShapeDtypeStruct((N, C, W), jnp.float32),
        compiler_params=pltpu.CompilerParams(use_tc_tiling_on_sc=True),
        scratch_types=[
            pltpu.VMEM((SPW, NLANES), jnp.int32),
            pltpu.VMEM((SPW, 2, NLANES), jnp.float32),
        ]
        + [pltpu.VMEM((2, CH, W), jnp.float32) for _ in range(NBUF)]
        + [pltpu.SemaphoreType.DMA for _ in range(2 * NBUF)],
    )
    def sc_kernel(x_hbm, ih_hbm, wts_hbm, out_hbm,
                  ih_v, w_v, b0, b1, b2, b3,
                  g0, g1, g2, g3, o0, o1, o2, o3):
        wid = lax.axis_index("s") * NCORES + lax.axis_index("c")
        base = wid * SPW
        pltpu.sync_copy(ih_hbm.at[pl.ds(base, SPW)], ih_v)
        pltpu.sync_copy(wts_hbm.at[pl.ds(base, SPW)], w_v)

        bufs = (b0, b1, b2, b3)
        gsems = (g0, g1, g2, g3)
        osems = (o0, o1, o2, o3)

        def taps(t):
            s = t // CSPLIT
            return s, ih_v[s]

        def start_gather(t, par, ch):
            s, v = taps(t)
            buf = bufs[par]
            pltpu.async_copy(
                x_hbm.at[v[0], pl.ds(ch * CH, CH), v[1], :], buf.at[0],
                gsems[par])
            pltpu.async_copy(
                x_hbm.at[v[2], pl.ds(ch * CH, CH), v[3], :], buf.at[1],
                gsems[par])

        def wait_gather(t, par, ch):
            s, v = taps(t)
            buf = bufs[par]
            pltpu.make_async_copy(
                x_hbm.at[v[0], pl.ds(ch * CH, CH), v[1], :], buf.at[0],
                gsems[par]).wait()
            pltpu.make_async_copy(
                x_hbm.at[v[2], pl.ds(ch * CH, CH), v[3], :], buf.at[1],
                gsems[par]).wait()

        def blend(t, par):
            s, _ = taps(t)
            buf = bufs[par]
            w0 = w_v[s, 0]
            w1 = w_v[s, 1]

            def rbody(r, _):
                for cc in range(W // NLANES):
                    sl = pl.ds(cc * NLANES, NLANES)
                    buf[0, r, sl] = w0 * buf[0, r, sl] + w1 * buf[1, r, sl]
                return 0

            lax.fori_loop(0, CH, rbody, 0)

        def start_writeout(t, par, ch):
            s, _ = taps(t)
            pltpu.async_copy(
                bufs[par].at[0],
                out_hbm.at[base + s, pl.ds(ch * CH, CH), :], osems[par])

        def wait_writeout(t, par, ch):
            s, _ = taps(t)
            pltpu.make_async_copy(
                bufs[par].at[0],
                out_hbm.at[base + s, pl.ds(ch * CH, CH), :],
                osems[par]).wait()

        start_gather(0, 0, 0)
        start_gather(1, 1, 1)
        for t in (0, 1):
            wait_gather(t, t, t)
            start_gather(t + 2, t + 2, t)
            blend(t, t)
            start_writeout(t, t, t)

        def group(k, _):
            g = 2 + k * NBUF
            for b in range(NBUF):
                t = g + b
                par = (2 + b) % NBUF
                ch = b % CSPLIT
                wait_gather(t, par, ch)
                wait_writeout(t - 2, (par + 2) % NBUF, ch)
                start_gather(t + 2, (par + 2) % NBUF, ch)
                blend(t, par)
                start_writeout(t, par, ch)
            return 0

        lax.fori_loop(0, (NT - 4) // NBUF, group, 0)

        for t in (NT - 2, NT - 1):
            par = t % NBUF
            ch = t % CSPLIT
            wait_gather(t, par, ch)
            blend(t, par)
            start_writeout(t, par, ch)
        for t in (NT - 4, NT - 3, NT - 2, NT - 1):
            wait_writeout(t, t % NBUF, t % CSPLIT)

    return sc_kernel


def kernel(x, image_num, image_ids, rows):
    del image_num, image_ids
    NIMG, C, H, W = x.shape
    NROW = NIMG * H

    iy = rows - 0.5
    iy0 = jnp.floor(iy)
    w1 = iy - iy0
    w0 = 1.0 - w1
    y0 = iy0.astype(jnp.int32)
    y1 = y0 + 1
    w0 = jnp.where((y0 >= 0) & (y0 <= NROW - 1), 0.5 * w0, 0.0)
    w1 = jnp.where((y1 >= 0) & (y1 <= NROW - 1), 0.5 * w1, 0.0)
    y0c = jnp.clip(y0, 0, NROW - 1)
    y1c = jnp.clip(y1, 0, NROW - 1)

    ih = jnp.stack(
        [y0c // H, y0c % H, y1c // H, y1c % H], axis=-1)
    ih = jnp.pad(ih, ((0, 0), (0, NLANES - 4)))
    wts = jnp.broadcast_to(
        jnp.stack([w0, w1], axis=1)[:, :, None], (N, 2, NLANES))

    return _make_sc_kernel(NIMG, C, H, W)(x, ih, wts)


# speedup vs baseline: 5.8199x; 1.2266x over previous
import functools

import jax
import jax.numpy as jnp
from jax import lax
from jax.experimental import pallas as pl
from jax.experimental.pallas import tpu as pltpu
from jax.experimental.pallas import tpu_sc as plsc

N = 1024
NLANES = 16
NCORES = 2
NSUBCORES = 16
NW = NCORES * NSUBCORES
SPW = N // NW
CSPLIT = 2
NBUF = 3
NOBUF = 2


def _make_sc_kernel(NIMG, C, H, W):
    CH = C // CSPLIT
    NT = SPW * CSPLIT
    mesh = plsc.VectorSubcoreMesh(core_axis_name="c", subcore_axis_name="s")

    @functools.partial(
        pl.kernel,
        mesh=mesh,
        out_type=jax.ShapeDtypeStruct((N, C, W), jnp.float32),
        compiler_params=pltpu.CompilerParams(use_tc_tiling_on_sc=True),
        scratch_types=[
            pltpu.VMEM((SPW, NLANES), jnp.int32),
            pltpu.VMEM((SPW, 2, NLANES), jnp.float32),
        ]
        + [pltpu.VMEM((2, CH, W), jnp.float32) for _ in range(NBUF)]
        + [pltpu.VMEM((CH, W), jnp.float32) for _ in range(NOBUF)]
        + [pltpu.SemaphoreType.DMA for _ in range(NBUF + NOBUF)],
    )
    def sc_kernel(x_hbm, ih_hbm, wts_hbm, out_hbm,
                  ih_v, w_v, b0, b1, b2, ob0, ob1,
                  g0, g1, g2, o0, o1):
        wid = lax.axis_index("s") * NCORES + lax.axis_index("c")
        base = wid * SPW
        pltpu.sync_copy(ih_hbm.at[pl.ds(base, SPW)], ih_v)
        pltpu.sync_copy(wts_hbm.at[pl.ds(base, SPW)], w_v)

        bufs = (b0, b1, b2)
        obufs = (ob0, ob1)
        gsems = (g0, g1, g2)
        osems = (o0, o1)

        def taps(t):
            s = t // CSPLIT
            return s, ih_v[s]

        def start_gather(t, par, ch):
            s, v = taps(t)
            buf = bufs[par]
            pltpu.async_copy(
                x_hbm.at[v[0], pl.ds(ch * CH, CH), v[1], :], buf.at[0],
                gsems[par])
            pltpu.async_copy(
                x_hbm.at[v[2], pl.ds(ch * CH, CH), v[3], :], buf.at[1],
                gsems[par])

        def wait_gather(t, par, ch):
            s, v = taps(t)
            buf = bufs[par]
            pltpu.make_async_copy(
                x_hbm.at[v[0], pl.ds(ch * CH, CH), v[1], :], buf.at[0],
                gsems[par]).wait()
            pltpu.make_async_copy(
                x_hbm.at[v[2], pl.ds(ch * CH, CH), v[3], :], buf.at[1],
                gsems[par]).wait()

        def blend(t, par, opar):
            s, _ = taps(t)
            buf = bufs[par]
            ob = obufs[opar]
            w0 = w_v[s, 0]
            w1 = w_v[s, 1]

            @plsc.parallel_loop(0, CH, unroll=2)
            def rbody(r):
                for cc in range(W // NLANES):
                    sl = pl.ds(cc * NLANES, NLANES)
                    ob[r, sl] = w0 * buf[0, r, sl] + w1 * buf[1, r, sl]

        def start_writeout(t, opar, ch):
            s, _ = taps(t)
            pltpu.async_copy(
                obufs[opar],
                out_hbm.at[base + s, pl.ds(ch * CH, CH), :], osems[opar])

        def wait_writeout(t, opar, ch):
            s, _ = taps(t)
            pltpu.make_async_copy(
                obufs[opar],
                out_hbm.at[base + s, pl.ds(ch * CH, CH), :],
                osems[opar]).wait()

        start_gather(0, 0, 0)
        start_gather(1, 1, 1)
        for t in (0, 1):
            wait_gather(t, t, t)
            start_gather(t + 2, (t + 2) % NBUF, t)
            blend(t, t, t)
            start_writeout(t, t, t)

        GRP = 6
        def group(k, _):
            g = 2 + k * GRP
            for b in range(GRP):
                t = g + b
                par = (2 + b) % NBUF
                opar = b % NOBUF
                ch = b % CSPLIT
                wait_gather(t, par, ch)
                start_gather(t + 2, (par + 2) % NBUF, ch)
                wait_writeout(t - 2, opar, ch)
                blend(t, par, opar)
                start_writeout(t, opar, ch)
            return 0

        lax.fori_loop(0, (NT - 4) // GRP, group, 0)

        for t in (NT - 2, NT - 1):
            par = t % NBUF
            opar = t % NOBUF
            ch = t % CSPLIT
            wait_gather(t, par, ch)
            wait_writeout(t - 2, opar, ch)
            blend(t, par, opar)
            start_writeout(t, opar, ch)
        for t in (NT - 2, NT - 1):
            wait_writeout(t, t % NOBUF, t % CSPLIT)

    return sc_kernel


def kernel(x, image_num, image_ids, rows):
    del image_num, image_ids
    NIMG, C, H, W = x.shape
    NROW = NIMG * H

    iy = rows - 0.5
    iy0 = jnp.floor(iy)
    w1 = iy - iy0
    w0 = 1.0 - w1
    y0 = iy0.astype(jnp.int32)
    y1 = y0 + 1
    w0 = jnp.where((y0 >= 0) & (y0 <= NROW - 1), 0.5 * w0, 0.0)
    w1 = jnp.where((y1 >= 0) & (y1 <= NROW - 1), 0.5 * w1, 0.0)
    y0c = jnp.clip(y0, 0, NROW - 1)
    y1c = jnp.clip(y1, 0, NROW - 1)

    ih = jnp.stack(
        [y0c // H, y0c % H, y1c // H, y1c % H], axis=-1)
    ih = jnp.pad(ih, ((0, 0), (0, NLANES - 4)))
    wts = jnp.broadcast_to(
        jnp.stack([w0, w1], axis=1)[:, :, None], (N, 2, NLANES))

    return _make_sc_kernel(NIMG, C, H, W)(x, ih, wts)
